# baseline (speedup 1.0000x reference)
import functools

import jax
import jax.numpy as jnp
from jax import lax
from jax.experimental import pallas as pl
from jax.experimental.pallas import tpu as pltpu
from jax.experimental.pallas import tpu_sc as plsc

N = 10000
E = 320000
NP = N + 112
B = 128
NBATCH = 2560
E_PAD = NBATCH * B
WB = NBATCH // 32
RPT = NP // 16
BN = 1000
ZSTARTS = (0, 128, 256, 384, 504)


def _zero_vmem(ref, nrows, ncols):
    z = jnp.zeros((16,), jnp.float32)

    def body(r, carry):
        for j in range(ncols // 16):
            ref[r, pl.ds(j * 16, 16)] = z
        return carry

    lax.fori_loop(0, nrows, body, 0)


_MESH = plsc.VectorSubcoreMesh(core_axis_name="c", subcore_axis_name="s")


@functools.partial(
    pl.kernel,
    out_type=jax.ShapeDtypeStruct((2 * NP, 16), jnp.float32),
    mesh=_MESH,
    scratch_types=(
        [pltpu.VMEM_SHARED((NP, 16), jnp.float32),
         pltpu.VMEM((B, 16), jnp.float32)]
        + [pltpu.VMEM((B,), jnp.int32)] * 6
        + [pltpu.SemaphoreType.DMA] * 8
    ),
)
def _deg_kernel(dst_hbm, deg_hbm, deg_sh, ones_v,
                d0, d1, d2, d3, d4, d5,
                ss0, ss1, is0, is1, is2, is3, is4, is5):
    c = lax.axis_index("c")
    t = lax.axis_index("s")
    base = t * RPT
    dsts = (d0, d1, d2, d3, d4, d5)
    ssems = (ss0, ss1)
    isems = (is0, is1, is2, is3, is4, is5)
    _zero_vmem(ones_v, B, 16)
    for st in ZSTARTS:
        pltpu.async_copy(ones_v, deg_sh.at[pl.ds(base + st, B)], ss0)
    for st in ZSTARTS:
        pltpu.make_async_copy(ones_v, deg_sh.at[pl.ds(base + st, B)],
                              ss0).wait()
    one = jnp.full((16,), 1.0, jnp.float32)

    def fill(r, carry):
        ones_v[r, pl.ds(0, 16)] = one
        return carry

    lax.fori_loop(0, B, fill, 0)
    plsc.subcore_barrier()

    first = (c * 16 + t) * WB

    def start_idx(k, u):
        pltpu.async_copy(dst_hbm.at[first + k], dsts[u], isems[u])

    def wait_idx(k, u):
        pltpu.make_async_copy(dst_hbm.at[first + k], dsts[u],
                              isems[u]).wait()

    def start_scatter(ki, u):
        pltpu.async_copy(ones_v, deg_sh.at[dsts[ki]], ssems[u], add=True)

    def wait_scatter(ki, u):
        pltpu.make_async_copy(ones_v, deg_sh.at[dsts[ki]], ssems[u]).wait()

    start_idx(0, 0)
    start_idx(1, 1)
    wait_idx(0, 0)
    start_scatter(0, 0)
    start_idx(2, 2)
    wait_idx(1, 1)
    start_scatter(1, 1)
    start_idx(3, 3)

    def body(j, carry):
        for u in range(6):
            k = 2 + 6 * j + u
            i6 = (2 + u) % 6
            r2 = u % 2
            wait_idx(k, i6)
            wait_scatter(u, r2)
            start_scatter(i6, r2)
            start_idx(k + 2, (4 + u) % 6)
        return carry

    lax.fori_loop(0, (WB - 2) // 6, body, 0)

    wait_scatter((WB - 2) % 6, 0)
    wait_scatter((WB - 1) % 6, 1)
    wait_idx(WB, WB % 6)
    wait_idx(WB + 1, (WB + 1) % 6)

    plsc.subcore_barrier()
    pltpu.sync_copy(deg_sh.at[pl.ds(base, RPT)],
                    deg_hbm.at[pl.ds(c * NP + base, RPT)])


@functools.partial(
    pl.kernel,
    out_type=jax.ShapeDtypeStruct((2 * NP, 128), jnp.float32),
    mesh=_MESH,
    scratch_types=(
        [pltpu.VMEM_SHARED((NP, 128), jnp.float32)]
        + [pltpu.VMEM((B,), jnp.int32)] * 12
        + [pltpu.VMEM((B, 128), jnp.float32)] * 2
        + [pltpu.SemaphoreType.DMA] * 10
    ),
)
def _edge_pass(g_hbm, src_hbm, dst_hbm, raw_hbm, acc_sh,
               s0, s1, s2, s3, s4, s5, d0, d1, d2, d3, d4, d5, rw0, rw1,
               gsem0, gsem1, ssem0, ssem1, is0, is1, is2, is3, is4, is5):
    c = lax.axis_index("c")
    t = lax.axis_index("s")
    base = t * RPT
    first = (c * 16 + t) * WB
    srcs = (s0, s1, s2, s3, s4, s5)
    dsts = (d0, d1, d2, d3, d4, d5)
    rows = (rw0, rw1)
    gsems = (gsem0, gsem1)
    ssems = (ssem0, ssem1)
    isems = (is0, is1, is2, is3, is4, is5)

    _zero_vmem(rw0, B, 128)
    for st in ZSTARTS:
        pltpu.async_copy(rw0, acc_sh.at[pl.ds(base + st, B)], gsem0)
    for st in ZSTARTS:
        pltpu.make_async_copy(rw0, acc_sh.at[pl.ds(base + st, B)],
                              gsem0).wait()
    plsc.subcore_barrier()

    def start_idx(k, u):
        pltpu.async_copy(src_hbm.at[first + k], srcs[u], isems[u])
        pltpu.async_copy(dst_hbm.at[first + k], dsts[u], isems[u])

    def wait_idx(k, u):
        pltpu.make_async_copy(src_hbm.at[first + k], srcs[u],
                              isems[u]).wait()
        pltpu.make_async_copy(dst_hbm.at[first + k], dsts[u],
                              isems[u]).wait()

    def start_gather(ki, u):
        pltpu.async_copy(g_hbm.at[srcs[ki]], rows[u], gsems[u])

    def wait_gather(ki, u):
        pltpu.make_async_copy(g_hbm.at[srcs[ki]], rows[u],
                              gsems[u]).wait()

    def start_scatter(ki, u):
        pltpu.async_copy(rows[u], acc_sh.at[dsts[ki]], ssems[u],
                         add=True)

    def wait_scatter(ki, u):
        pltpu.make_async_copy(rows[u], acc_sh.at[dsts[ki]],
                              ssems[u]).wait()

    start_idx(0, 0)
    start_idx(1, 1)
    wait_idx(0, 0)
    start_gather(0, 0)
    start_idx(2, 2)
    wait_idx(1, 1)
    start_gather(1, 1)
    start_idx(3, 3)
    wait_gather(0, 0)
    start_scatter(0, 0)

    def body(j, carry):
        for u in range(6):
            k = 2 + 6 * j + u
            i6 = (2 + u) % 6
            r2 = u % 2
            wait_idx(k, i6)
            wait_gather((1 + u) % 6, 1 - r2)
            start_scatter((1 + u) % 6, 1 - r2)
            wait_scatter(u, r2)
            start_gather(i6, r2)
            start_idx(k + 2, (4 + u) % 6)
        return carry

    lax.fori_loop(0, (WB - 2) // 6, body, 0)

    wait_gather((WB - 1) % 6, 1)
    start_scatter((WB - 1) % 6, 1)
    wait_scatter((WB - 2) % 6, 0)
    wait_scatter((WB - 1) % 6, 1)
    wait_idx(WB, WB % 6)
    wait_idx(WB + 1, (WB + 1) % 6)

    plsc.subcore_barrier()
    pltpu.sync_copy(acc_sh.at[pl.ds(base, RPT)],
                    raw_hbm.at[pl.ds(c * NP + base, RPT)])


def _dis_block(da_ref, db_ref):
    deg = da_ref[0, :, 0:1] + db_ref[0, :, 0:1] + 1.0
    return lax.rsqrt(deg)


def _tc_first(x, W1, deg3d):
    def body(x_ref, w_ref, da_ref, db_ref, o_ref):
        dis = _dis_block(da_ref, db_ref)
        h = jnp.dot(x_ref[...], w_ref[...], preferred_element_type=jnp.float32)
        o_ref[...] = dis * h

    nb = N // BN
    return pl.pallas_call(
        body,
        grid=(nb,),
        in_specs=[
            pl.BlockSpec((BN, 128), lambda i: (i, 0)),
            pl.BlockSpec((128, 128), lambda i: (0, 0)),
            pl.BlockSpec((1, BN, 16), lambda i: (0, i, 0)),
            pl.BlockSpec((1, BN, 16), lambda i: (1, i, 0)),
        ],
        out_specs=pl.BlockSpec((BN, 128), lambda i: (i, 0)),
        out_shape=jax.ShapeDtypeStruct((N, 128), jnp.float32),
    )(x, W1, deg3d, deg3d)


def _tc_mid(raw3d, g, deg3d, b2d, W):
    fo = W.shape[1]

    def body(ra_ref, rb_ref, g_ref, da_ref, db_ref, b_ref, w_ref, o_ref):
        dis = _dis_block(da_ref, db_ref)
        z = dis * (ra_ref[0] + rb_ref[0] + g_ref[...]) + b_ref[...]
        a = jnp.maximum(z, 0.0)
        h = jnp.dot(a, w_ref[...], preferred_element_type=jnp.float32)
        h = dis * h
        if fo < 128:
            h = jnp.concatenate(
                [h, jnp.zeros((h.shape[0], 128 - fo), jnp.float32)], axis=1)
        o_ref[...] = h

    nb = N // BN
    return pl.pallas_call(
        body,
        grid=(nb,),
        in_specs=[
            pl.BlockSpec((1, BN, 128), lambda i: (0, i, 0)),
            pl.BlockSpec((1, BN, 128), lambda i: (1, i, 0)),
            pl.BlockSpec((BN, 128), lambda i: (i, 0)),
            pl.BlockSpec((1, BN, 16), lambda i: (0, i, 0)),
            pl.BlockSpec((1, BN, 16), lambda i: (1, i, 0)),
            pl.BlockSpec((1, 128), lambda i: (0, 0)),
            pl.BlockSpec((128, fo), lambda i: (0, 0)),
        ],
        out_specs=pl.BlockSpec((BN, 128), lambda i: (i, 0)),
        out_shape=jax.ShapeDtypeStruct((N, 128), jnp.float32),
    )(raw3d, raw3d, g, deg3d, deg3d, b2d, W)


def _tc_final(raw3d, g, deg3d, b2):
    def body(ra_ref, rb_ref, g_ref, da_ref, db_ref, b_ref, o_ref):
        dis = _dis_block(da_ref, db_ref)
        z = dis * (ra_ref[0, :, 0:64] + rb_ref[0, :, 0:64] + g_ref[:, 0:64])
        z = z + b_ref[...]
        m = jnp.max(z, axis=1, keepdims=True)
        s = z - m
        o_ref[...] = s - jnp.log(jnp.sum(jnp.exp(s), axis=1, keepdims=True))

    nb = N // BN
    return pl.pallas_call(
        body,
        grid=(nb,),
        in_specs=[
            pl.BlockSpec((1, BN, 128), lambda i: (0, i, 0)),
            pl.BlockSpec((1, BN, 128), lambda i: (1, i, 0)),
            pl.BlockSpec((BN, 128), lambda i: (i, 0)),
            pl.BlockSpec((1, BN, 16), lambda i: (0, i, 0)),
            pl.BlockSpec((1, BN, 16), lambda i: (1, i, 0)),
            pl.BlockSpec((1, 64), lambda i: (0, 0)),
        ],
        out_specs=pl.BlockSpec((BN, 64), lambda i: (i, 0)),
        out_shape=jax.ShapeDtypeStruct((N, 64), jnp.float32),
    )(raw3d, raw3d, g, deg3d, deg3d, b2)


def kernel(x, edge_index, W1, b1, Wg0, bg0, Wg1, bg1, W2, b2):
    src = edge_index[0]
    dst = edge_index[1]
    pad = E_PAD - E + 2 * B
    ar = jnp.arange(pad, dtype=jnp.int32)
    srcp = jnp.concatenate([src, ar % N]).reshape(NBATCH + 2, B)
    dstp = jnp.concatenate([dst, N + (ar % 112)]).reshape(NBATCH + 2, B)

    deg3d = _deg_kernel(dstp).reshape(2, NP, 16)

    g1 = _tc_first(x, W1, deg3d)
    raw1 = _edge_pass(g1, srcp, dstp).reshape(2, NP, 128)
    g2 = _tc_mid(raw1, g1, deg3d, b1.reshape(1, 128), Wg0)
    raw2 = _edge_pass(g2, srcp, dstp).reshape(2, NP, 128)
    g3 = _tc_mid(raw2, g2, deg3d, bg0.reshape(1, 128), Wg1)
    raw3 = _edge_pass(g3, srcp, dstp).reshape(2, NP, 128)
    g4 = _tc_mid(raw3, g3, deg3d, bg1.reshape(1, 128), W2)
    raw4 = _edge_pass(g4, srcp, dstp).reshape(2, NP, 128)
    return _tc_final(raw4, g4, deg3d, b2.reshape(1, 64))


# speedup vs baseline: 1.1564x; 1.1564x over previous
import functools

import jax
import jax.numpy as jnp
from jax import lax
from jax.experimental import pallas as pl
from jax.experimental.pallas import tpu as pltpu
from jax.experimental.pallas import tpu_sc as plsc

N = 10000
E = 320000
NP = N + 112
B = 128
NBATCH = 2560
E_PAD = NBATCH * B
WB = NBATCH // 32
RPT = NP // 16
BN = 1000
ZSTARTS = (0, 128, 256, 384, 504)


def _zero_vmem(ref, nrows, ncols):
    z = jnp.zeros((16,), jnp.float32)

    def body(r, carry):
        for j in range(ncols // 16):
            ref[r, pl.ds(j * 16, 16)] = z
        return carry

    lax.fori_loop(0, nrows, body, 0)


_MESH = plsc.VectorSubcoreMesh(core_axis_name="c", subcore_axis_name="s")


@functools.partial(
    pl.kernel,
    out_type=jax.ShapeDtypeStruct((2 * NP, 16), jnp.float32),
    mesh=_MESH,
    scratch_types=(
        [pltpu.VMEM_SHARED((NP, 16), jnp.float32),
         pltpu.VMEM((B, 16), jnp.float32)]
        + [pltpu.VMEM((B,), jnp.int32)] * 6
        + [pltpu.SemaphoreType.DMA] * 8
    ),
)
def _deg_kernel(dst_hbm, deg_hbm, deg_sh, ones_v,
                d0, d1, d2, d3, d4, d5,
                ss0, ss1, is0, is1, is2, is3, is4, is5):
    c = lax.axis_index("c")
    t = lax.axis_index("s")
    base = t * RPT
    dsts = (d0, d1, d2, d3, d4, d5)
    ssems = (ss0, ss1)
    isems = (is0, is1, is2, is3, is4, is5)
    _zero_vmem(ones_v, B, 16)
    for st in ZSTARTS:
        pltpu.async_copy(ones_v, deg_sh.at[pl.ds(base + st, B)], ss0)
    for st in ZSTARTS:
        pltpu.make_async_copy(ones_v, deg_sh.at[pl.ds(base + st, B)],
                              ss0).wait()
    one = jnp.full((16,), 1.0, jnp.float32)

    def fill(r, carry):
        ones_v[r, pl.ds(0, 16)] = one
        return carry

    lax.fori_loop(0, B, fill, 0)
    plsc.subcore_barrier()

    first = (c * 16 + t) * WB

    def start_idx(k, u):
        pltpu.async_copy(dst_hbm.at[first + k], dsts[u], isems[u])

    def wait_idx(k, u):
        pltpu.make_async_copy(dst_hbm.at[first + k], dsts[u],
                              isems[u]).wait()

    def start_scatter(ki, u):
        pltpu.async_copy(ones_v, deg_sh.at[dsts[ki]], ssems[u], add=True)

    def wait_scatter(ki, u):
        pltpu.make_async_copy(ones_v, deg_sh.at[dsts[ki]], ssems[u]).wait()

    start_idx(0, 0)
    start_idx(1, 1)
    wait_idx(0, 0)
    start_scatter(0, 0)
    start_idx(2, 2)
    wait_idx(1, 1)
    start_scatter(1, 1)
    start_idx(3, 3)

    def body(j, carry):
        for u in range(6):
            k = 2 + 6 * j + u
            i6 = (2 + u) % 6
            r2 = u % 2
            wait_idx(k, i6)
            wait_scatter(u, r2)
            start_scatter(i6, r2)
            start_idx(k + 2, (4 + u) % 6)
        return carry

    lax.fori_loop(0, (WB - 2) // 6, body, 0)

    wait_scatter((WB - 2) % 6, 0)
    wait_scatter((WB - 1) % 6, 1)
    wait_idx(WB, WB % 6)
    wait_idx(WB + 1, (WB + 1) % 6)

    plsc.subcore_barrier()
    pltpu.sync_copy(deg_sh.at[pl.ds(base, RPT)],
                    deg_hbm.at[pl.ds(c * NP + base, RPT)])


@functools.partial(
    pl.kernel,
    out_type=jax.ShapeDtypeStruct((2 * NP, 128), jnp.float32),
    mesh=_MESH,
    scratch_types=(
        [pltpu.VMEM_SHARED((NP, 128), jnp.float32)]
        + [pltpu.VMEM((B,), jnp.int32)] * 12
        + [pltpu.VMEM((B, 128), jnp.float32)] * 2
        + [pltpu.SemaphoreType.DMA] * 10
    ),
)
def _edge_pass(g_hbm, src_hbm, dst_hbm, raw_hbm, acc_sh,
               s0, s1, s2, s3, s4, s5, d0, d1, d2, d3, d4, d5, rw0, rw1,
               gsem0, gsem1, ssem0, ssem1, is0, is1, is2, is3, is4, is5):
    c = lax.axis_index("c")
    t = lax.axis_index("s")
    base = t * RPT
    first = (c * 16 + t) * WB
    srcs = (s0, s1, s2, s3, s4, s5)
    dsts = (d0, d1, d2, d3, d4, d5)
    rows = (rw0, rw1)
    gsems = (gsem0, gsem1)
    ssems = (ssem0, ssem1)
    isems = (is0, is1, is2, is3, is4, is5)

    _zero_vmem(rw0, B, 128)
    for st in ZSTARTS:
        pltpu.async_copy(rw0, acc_sh.at[pl.ds(base + st, B)], gsem0)
    for st in ZSTARTS:
        pltpu.make_async_copy(rw0, acc_sh.at[pl.ds(base + st, B)],
                              gsem0).wait()
    plsc.subcore_barrier()

    def start_idx(k, u):
        pltpu.async_copy(src_hbm.at[first + k], srcs[u], isems[u])
        pltpu.async_copy(dst_hbm.at[first + k], dsts[u], isems[u])

    def wait_idx(k, u):
        pltpu.make_async_copy(src_hbm.at[first + k], srcs[u],
                              isems[u]).wait()
        pltpu.make_async_copy(dst_hbm.at[first + k], dsts[u],
                              isems[u]).wait()

    def start_gather(ki, u):
        pltpu.async_copy(g_hbm.at[srcs[ki]], rows[u], gsems[u])

    def wait_gather(ki, u):
        pltpu.make_async_copy(g_hbm.at[srcs[ki]], rows[u],
                              gsems[u]).wait()

    def start_scatter(ki, u):
        pltpu.async_copy(rows[u], acc_sh.at[dsts[ki]], ssems[u],
                         add=True)

    def wait_scatter(ki, u):
        pltpu.make_async_copy(rows[u], acc_sh.at[dsts[ki]],
                              ssems[u]).wait()

    start_idx(0, 0)
    start_idx(1, 1)
    wait_idx(0, 0)
    start_gather(0, 0)
    start_idx(2, 2)
    wait_idx(1, 1)
    start_gather(1, 1)
    start_idx(3, 3)
    wait_gather(0, 0)
    start_scatter(0, 0)

    def body(j, carry):
        for u in range(6):
            k = 2 + 6 * j + u
            i6 = (2 + u) % 6
            r2 = u % 2
            wait_idx(k, i6)
            wait_scatter(u, r2)
            start_gather(i6, r2)
            start_idx(k + 2, (4 + u) % 6)
            wait_gather((1 + u) % 6, 1 - r2)
            start_scatter((1 + u) % 6, 1 - r2)
        return carry

    lax.fori_loop(0, (WB - 2) // 6, body, 0)

    wait_gather((WB - 1) % 6, 1)
    start_scatter((WB - 1) % 6, 1)
    wait_scatter((WB - 2) % 6, 0)
    wait_scatter((WB - 1) % 6, 1)
    wait_idx(WB, WB % 6)
    wait_idx(WB + 1, (WB + 1) % 6)

    plsc.subcore_barrier()
    pltpu.sync_copy(acc_sh.at[pl.ds(base, RPT)],
                    raw_hbm.at[pl.ds(c * NP + base, RPT)])


def _dis_block(da_ref, db_ref):
    deg = da_ref[0, :, 0:1] + db_ref[0, :, 0:1] + 1.0
    return lax.rsqrt(deg)


def _tc_first(x, W1, deg3d):
    def body(x_ref, w_ref, da_ref, db_ref, o_ref):
        dis = _dis_block(da_ref, db_ref)
        h = jnp.dot(x_ref[...], w_ref[...], preferred_element_type=jnp.float32)
        o_ref[...] = dis * h

    nb = N // BN
    return pl.pallas_call(
        body,
        grid=(nb,),
        in_specs=[
            pl.BlockSpec((BN, 128), lambda i: (i, 0)),
            pl.BlockSpec((128, 128), lambda i: (0, 0)),
            pl.BlockSpec((1, BN, 16), lambda i: (0, i, 0)),
            pl.BlockSpec((1, BN, 16), lambda i: (1, i, 0)),
        ],
        out_specs=pl.BlockSpec((BN, 128), lambda i: (i, 0)),
        out_shape=jax.ShapeDtypeStruct((N, 128), jnp.float32),
    )(x, W1, deg3d, deg3d)


def _tc_mid(raw3d, g, deg3d, b2d, W):
    fo = W.shape[1]

    def body(ra_ref, rb_ref, g_ref, da_ref, db_ref, b_ref, w_ref, o_ref):
        dis = _dis_block(da_ref, db_ref)
        z = dis * (ra_ref[0] + rb_ref[0] + g_ref[...]) + b_ref[...]
        a = jnp.maximum(z, 0.0)
        h = jnp.dot(a, w_ref[...], preferred_element_type=jnp.float32)
        h = dis * h
        if fo < 128:
            h = jnp.concatenate(
                [h, jnp.zeros((h.shape[0], 128 - fo), jnp.float32)], axis=1)
        o_ref[...] = h

    nb = N // BN
    return pl.pallas_call(
        body,
        grid=(nb,),
        in_specs=[
            pl.BlockSpec((1, BN, 128), lambda i: (0, i, 0)),
            pl.BlockSpec((1, BN, 128), lambda i: (1, i, 0)),
            pl.BlockSpec((BN, 128), lambda i: (i, 0)),
            pl.BlockSpec((1, BN, 16), lambda i: (0, i, 0)),
            pl.BlockSpec((1, BN, 16), lambda i: (1, i, 0)),
            pl.BlockSpec((1, 128), lambda i: (0, 0)),
            pl.BlockSpec((128, fo), lambda i: (0, 0)),
        ],
        out_specs=pl.BlockSpec((BN, 128), lambda i: (i, 0)),
        out_shape=jax.ShapeDtypeStruct((N, 128), jnp.float32),
    )(raw3d, raw3d, g, deg3d, deg3d, b2d, W)


def _tc_final(raw3d, g, deg3d, b2):
    def body(ra_ref, rb_ref, g_ref, da_ref, db_ref, b_ref, o_ref):
        dis = _dis_block(da_ref, db_ref)
        z = dis * (ra_ref[0, :, 0:64] + rb_ref[0, :, 0:64] + g_ref[:, 0:64])
        z = z + b_ref[...]
        m = jnp.max(z, axis=1, keepdims=True)
        s = z - m
        o_ref[...] = s - jnp.log(jnp.sum(jnp.exp(s), axis=1, keepdims=True))

    nb = N // BN
    return pl.pallas_call(
        body,
        grid=(nb,),
        in_specs=[
            pl.BlockSpec((1, BN, 128), lambda i: (0, i, 0)),
            pl.BlockSpec((1, BN, 128), lambda i: (1, i, 0)),
            pl.BlockSpec((BN, 128), lambda i: (i, 0)),
            pl.BlockSpec((1, BN, 16), lambda i: (0, i, 0)),
            pl.BlockSpec((1, BN, 16), lambda i: (1, i, 0)),
            pl.BlockSpec((1, 64), lambda i: (0, 0)),
        ],
        out_specs=pl.BlockSpec((BN, 64), lambda i: (i, 0)),
        out_shape=jax.ShapeDtypeStruct((N, 64), jnp.float32),
    )(raw3d, raw3d, g, deg3d, deg3d, b2)


def kernel(x, edge_index, W1, b1, Wg0, bg0, Wg1, bg1, W2, b2):
    src = edge_index[0]
    dst = edge_index[1]
    pad = E_PAD - E + 2 * B
    ar = jnp.arange(pad, dtype=jnp.int32)
    srcp = jnp.concatenate([src, ar % N]).reshape(NBATCH + 2, B)
    dstp = jnp.concatenate([dst, N + (ar % 112)]).reshape(NBATCH + 2, B)

    deg3d = _deg_kernel(dstp).reshape(2, NP, 16)

    g1 = _tc_first(x, W1, deg3d)
    raw1 = _edge_pass(g1, srcp, dstp).reshape(2, NP, 128)
    g2 = _tc_mid(raw1, g1, deg3d, b1.reshape(1, 128), Wg0)
    raw2 = _edge_pass(g2, srcp, dstp).reshape(2, NP, 128)
    g3 = _tc_mid(raw2, g2, deg3d, bg0.reshape(1, 128), Wg1)
    raw3 = _edge_pass(g3, srcp, dstp).reshape(2, NP, 128)
    g4 = _tc_mid(raw3, g3, deg3d, bg1.reshape(1, 128), W2)
    raw4 = _edge_pass(g4, srcp, dstp).reshape(2, NP, 128)
    return _tc_final(raw4, g4, deg3d, b2.reshape(1, 64))


# speedup vs baseline: 1.2523x; 1.0830x over previous
import functools

import jax
import jax.numpy as jnp
from jax import lax
from jax.experimental import pallas as pl
from jax.experimental.pallas import tpu as pltpu
from jax.experimental.pallas import tpu_sc as plsc

N = 10000
E = 320000
NP = N + 112
B = 120
NBATCH = 2752
E_PAD = NBATCH * B
WB = NBATCH // 32
RPT = NP // 16
BN = 1000
ZSTARTS = (0, 120, 240, 360, 480, 512)


def _zero_vmem(ref, nrows, ncols):
    z = jnp.zeros((16,), jnp.float32)

    def body(r, carry):
        for j in range(ncols // 16):
            ref[r, pl.ds(j * 16, 16)] = z
        return carry

    lax.fori_loop(0, nrows, body, 0)


_MESH = plsc.VectorSubcoreMesh(core_axis_name="c", subcore_axis_name="s")


@functools.partial(
    pl.kernel,
    out_type=jax.ShapeDtypeStruct((2 * NP, 16), jnp.float32),
    mesh=_MESH,
    scratch_types=(
        [pltpu.VMEM_SHARED((NP, 16), jnp.float32),
         pltpu.VMEM((B, 16), jnp.float32)]
        + [pltpu.VMEM((B,), jnp.int32)] * 6
        + [pltpu.SemaphoreType.DMA] * 8
    ),
)
def _deg_kernel(dst_hbm, deg_hbm, deg_sh, ones_v,
                d0, d1, d2, d3, d4, d5,
                ss0, ss1, is0, is1, is2, is3, is4, is5):
    c = lax.axis_index("c")
    t = lax.axis_index("s")
    base = t * RPT
    dsts = (d0, d1, d2, d3, d4, d5)
    ssems = (ss0, ss1)
    isems = (is0, is1, is2, is3, is4, is5)
    _zero_vmem(ones_v, B, 16)
    for st in ZSTARTS:
        pltpu.async_copy(ones_v, deg_sh.at[pl.ds(base + st, B)], ss0)
    for st in ZSTARTS:
        pltpu.make_async_copy(ones_v, deg_sh.at[pl.ds(base + st, B)],
                              ss0).wait()
    one = jnp.full((16,), 1.0, jnp.float32)

    def fill(r, carry):
        ones_v[r, pl.ds(0, 16)] = one
        return carry

    lax.fori_loop(0, B, fill, 0)
    plsc.subcore_barrier()

    first = (c * 16 + t) * WB

    def start_idx(k, u):
        pltpu.async_copy(dst_hbm.at[first + k], dsts[u], isems[u])

    def wait_idx(k, u):
        pltpu.make_async_copy(dst_hbm.at[first + k], dsts[u],
                              isems[u]).wait()

    def start_scatter(ki, u):
        pltpu.async_copy(ones_v, deg_sh.at[dsts[ki]], ssems[u], add=True)

    def wait_scatter(ki, u):
        pltpu.make_async_copy(ones_v, deg_sh.at[dsts[ki]], ssems[u]).wait()

    start_idx(0, 0)
    start_idx(1, 1)
    wait_idx(0, 0)
    start_scatter(0, 0)
    start_idx(2, 2)
    wait_idx(1, 1)
    start_scatter(1, 1)
    start_idx(3, 3)

    def body(j, carry):
        for u in range(6):
            k = 2 + 6 * j + u
            i6 = (2 + u) % 6
            r2 = u % 2
            wait_idx(k, i6)
            wait_scatter(u, r2)
            start_scatter(i6, r2)
            start_idx(k + 2, (4 + u) % 6)
        return carry

    lax.fori_loop(0, (WB - 2) // 6, body, 0)

    wait_scatter((WB - 2) % 6, 0)
    wait_scatter((WB - 1) % 6, 1)
    wait_idx(WB, WB % 6)
    wait_idx(WB + 1, (WB + 1) % 6)

    plsc.subcore_barrier()
    pltpu.sync_copy(deg_sh.at[pl.ds(base, RPT)],
                    deg_hbm.at[pl.ds(c * NP + base, RPT)])


@functools.partial(
    pl.kernel,
    out_type=jax.ShapeDtypeStruct((2 * NP, 128), jnp.float32),
    mesh=_MESH,
    scratch_types=(
        [pltpu.VMEM_SHARED((NP, 128), jnp.float32)]
        + [pltpu.VMEM((B,), jnp.int32)] * 12
        + [pltpu.VMEM((B, 128), jnp.float32)] * 3
        + [pltpu.SemaphoreType.DMA] * 12
    ),
)
def _edge_pass(g_hbm, src_hbm, dst_hbm, raw_hbm, acc_sh,
               s0, s1, s2, s3, s4, s5, d0, d1, d2, d3, d4, d5, rw0, rw1, rw2,
               gsem0, gsem1, gsem2, ssem0, ssem1, ssem2,
               is0, is1, is2, is3, is4, is5):
    c = lax.axis_index("c")
    t = lax.axis_index("s")
    base = t * RPT
    first = (c * 16 + t) * WB
    srcs = (s0, s1, s2, s3, s4, s5)
    dsts = (d0, d1, d2, d3, d4, d5)
    rows = (rw0, rw1, rw2)
    gsems = (gsem0, gsem1, gsem2)
    ssems = (ssem0, ssem1, ssem2)
    isems = (is0, is1, is2, is3, is4, is5)

    _zero_vmem(rw0, B, 128)
    for st in ZSTARTS:
        pltpu.async_copy(rw0, acc_sh.at[pl.ds(base + st, B)], gsem0)
    for st in ZSTARTS:
        pltpu.make_async_copy(rw0, acc_sh.at[pl.ds(base + st, B)],
                              gsem0).wait()
    plsc.subcore_barrier()

    def start_idx(k, u):
        pltpu.async_copy(src_hbm.at[first + k], srcs[u], isems[u])
        pltpu.async_copy(dst_hbm.at[first + k], dsts[u], isems[u])

    def wait_idx(k, u):
        pltpu.make_async_copy(src_hbm.at[first + k], srcs[u],
                              isems[u]).wait()
        pltpu.make_async_copy(dst_hbm.at[first + k], dsts[u],
                              isems[u]).wait()

    def start_gather(ki, u):
        pltpu.async_copy(g_hbm.at[srcs[ki]], rows[u], gsems[u])

    def wait_gather(ki, u):
        pltpu.make_async_copy(g_hbm.at[srcs[ki]], rows[u],
                              gsems[u]).wait()

    def start_scatter(ki, u):
        pltpu.async_copy(rows[u], acc_sh.at[dsts[ki]], ssems[u],
                         add=True)

    def wait_scatter(ki, u):
        pltpu.make_async_copy(rows[u], acc_sh.at[dsts[ki]],
                              ssems[u]).wait()

    start_idx(0, 0)
    start_idx(1, 1)
    wait_idx(0, 0)
    start_gather(0, 0)
    start_idx(2, 2)
    wait_idx(1, 1)
    start_gather(1, 1)
    start_idx(3, 3)
    wait_gather(0, 0)
    start_scatter(0, 0)

    def body(j, carry):
        for u in range(6):
            k = 2 + 6 * j + u
            i6 = (2 + u) % 6
            r3 = (2 + u) % 3
            wait_idx(k, i6)
            start_gather(i6, r3)
            start_idx(k + 2, (4 + u) % 6)
            wait_gather((1 + u) % 6, (1 + u) % 3)
            wait_scatter(u % 6, u % 3)
            start_scatter((1 + u) % 6, (1 + u) % 3)
        return carry

    lax.fori_loop(0, (WB - 2) // 6, body, 0)

    wait_gather((WB - 1) % 6, (WB - 1) % 3)
    wait_scatter((WB - 2) % 6, (WB - 2) % 3)
    start_scatter((WB - 1) % 6, (WB - 1) % 3)
    wait_scatter((WB - 1) % 6, (WB - 1) % 3)
    wait_idx(WB, WB % 6)
    wait_idx(WB + 1, (WB + 1) % 6)

    plsc.subcore_barrier()
    pltpu.sync_copy(acc_sh.at[pl.ds(base, RPT)],
                    raw_hbm.at[pl.ds(c * NP + base, RPT)])


def _dis_block(da_ref, db_ref):
    deg = da_ref[0, :, 0:1] + db_ref[0, :, 0:1] + 1.0
    return lax.rsqrt(deg)


def _tc_first(x, W1, deg3d):
    def body(x_ref, w_ref, da_ref, db_ref, o_ref):
        dis = _dis_block(da_ref, db_ref)
        h = jnp.dot(x_ref[...], w_ref[...], preferred_element_type=jnp.float32)
        o_ref[...] = dis * h

    nb = N // BN
    return pl.pallas_call(
        body,
        grid=(nb,),
        in_specs=[
            pl.BlockSpec((BN, 128), lambda i: (i, 0)),
            pl.BlockSpec((128, 128), lambda i: (0, 0)),
            pl.BlockSpec((1, BN, 16), lambda i: (0, i, 0)),
            pl.BlockSpec((1, BN, 16), lambda i: (1, i, 0)),
        ],
        out_specs=pl.BlockSpec((BN, 128), lambda i: (i, 0)),
        out_shape=jax.ShapeDtypeStruct((N, 128), jnp.float32),
    )(x, W1, deg3d, deg3d)


def _tc_mid(raw3d, g, deg3d, b2d, W):
    fo = W.shape[1]

    def body(ra_ref, rb_ref, g_ref, da_ref, db_ref, b_ref, w_ref, o_ref):
        dis = _dis_block(da_ref, db_ref)
        z = dis * (ra_ref[0] + rb_ref[0] + g_ref[...]) + b_ref[...]
        a = jnp.maximum(z, 0.0)
        h = jnp.dot(a, w_ref[...], preferred_element_type=jnp.float32)
        h = dis * h
        if fo < 128:
            h = jnp.concatenate(
                [h, jnp.zeros((h.shape[0], 128 - fo), jnp.float32)], axis=1)
        o_ref[...] = h

    nb = N // BN
    return pl.pallas_call(
        body,
        grid=(nb,),
        in_specs=[
            pl.BlockSpec((1, BN, 128), lambda i: (0, i, 0)),
            pl.BlockSpec((1, BN, 128), lambda i: (1, i, 0)),
            pl.BlockSpec((BN, 128), lambda i: (i, 0)),
            pl.BlockSpec((1, BN, 16), lambda i: (0, i, 0)),
            pl.BlockSpec((1, BN, 16), lambda i: (1, i, 0)),
            pl.BlockSpec((1, 128), lambda i: (0, 0)),
            pl.BlockSpec((128, fo), lambda i: (0, 0)),
        ],
        out_specs=pl.BlockSpec((BN, 128), lambda i: (i, 0)),
        out_shape=jax.ShapeDtypeStruct((N, 128), jnp.float32),
    )(raw3d, raw3d, g, deg3d, deg3d, b2d, W)


def _tc_final(raw3d, g, deg3d, b2):
    def body(ra_ref, rb_ref, g_ref, da_ref, db_ref, b_ref, o_ref):
        dis = _dis_block(da_ref, db_ref)
        z = dis * (ra_ref[0, :, 0:64] + rb_ref[0, :, 0:64] + g_ref[:, 0:64])
        z = z + b_ref[...]
        m = jnp.max(z, axis=1, keepdims=True)
        s = z - m
        o_ref[...] = s - jnp.log(jnp.sum(jnp.exp(s), axis=1, keepdims=True))

    nb = N // BN
    return pl.pallas_call(
        body,
        grid=(nb,),
        in_specs=[
            pl.BlockSpec((1, BN, 128), lambda i: (0, i, 0)),
            pl.BlockSpec((1, BN, 128), lambda i: (1, i, 0)),
            pl.BlockSpec((BN, 128), lambda i: (i, 0)),
            pl.BlockSpec((1, BN, 16), lambda i: (0, i, 0)),
            pl.BlockSpec((1, BN, 16), lambda i: (1, i, 0)),
            pl.BlockSpec((1, 64), lambda i: (0, 0)),
        ],
        out_specs=pl.BlockSpec((BN, 64), lambda i: (i, 0)),
        out_shape=jax.ShapeDtypeStruct((N, 64), jnp.float32),
    )(raw3d, raw3d, g, deg3d, deg3d, b2)


def kernel(x, edge_index, W1, b1, Wg0, bg0, Wg1, bg1, W2, b2):
    src = edge_index[0]
    dst = edge_index[1]
    pad = E_PAD - E + 2 * B
    ar = jnp.arange(pad, dtype=jnp.int32)
    srcp = jnp.concatenate([src, ar % N]).reshape(NBATCH + 2, B)
    dstp = jnp.concatenate([dst, N + (ar % 112)]).reshape(NBATCH + 2, B)

    deg3d = _deg_kernel(dstp).reshape(2, NP, 16)

    g1 = _tc_first(x, W1, deg3d)
    raw1 = _edge_pass(g1, srcp, dstp).reshape(2, NP, 128)
    g2 = _tc_mid(raw1, g1, deg3d, b1.reshape(1, 128), Wg0)
    raw2 = _edge_pass(g2, srcp, dstp).reshape(2, NP, 128)
    g3 = _tc_mid(raw2, g2, deg3d, bg0.reshape(1, 128), Wg1)
    raw3 = _edge_pass(g3, srcp, dstp).reshape(2, NP, 128)
    g4 = _tc_mid(raw3, g3, deg3d, bg1.reshape(1, 128), W2)
    raw4 = _edge_pass(g4, srcp, dstp).reshape(2, NP, 128)
    return _tc_final(raw4, g4, deg3d, b2.reshape(1, 64))


# speedup vs baseline: 1.2711x; 1.0150x over previous
import functools

import jax
import jax.numpy as jnp
from jax import lax
from jax.experimental import pallas as pl
from jax.experimental.pallas import tpu as pltpu
from jax.experimental.pallas import tpu_sc as plsc

N = 10000
E = 320000
NP = N + 112
B = 120
NBATCH = 2752
E_PAD = NBATCH * B
WB = NBATCH // 32
RPT = NP // 16
BN = 1000
ZSTARTS = (0, 120, 240, 360, 480, 512)


def _zero_vmem(ref, nrows, ncols):
    z = jnp.zeros((16,), jnp.float32)

    def body(r, carry):
        for j in range(ncols // 16):
            ref[r, pl.ds(j * 16, 16)] = z
        return carry

    lax.fori_loop(0, nrows, body, 0)


_MESH = plsc.VectorSubcoreMesh(core_axis_name="c", subcore_axis_name="s")


@functools.partial(
    pl.kernel,
    out_type=jax.ShapeDtypeStruct((2 * NP, 16), jnp.float32),
    mesh=_MESH,
    scratch_types=(
        [pltpu.VMEM_SHARED((NP, 16), jnp.float32),
         pltpu.VMEM((B, 16), jnp.float32)]
        + [pltpu.VMEM((B,), jnp.int32)] * 6
        + [pltpu.SemaphoreType.DMA] * 12
    ),
)
def _deg_kernel(dst_hbm, deg_hbm, deg_sh, ones_v,
                d0, d1, d2, d3, d4, d5,
                ss0, ss1, ss2, ss3, ss4, ss5,
                is0, is1, is2, is3, is4, is5):
    c = lax.axis_index("c")
    t = lax.axis_index("s")
    base = t * RPT
    dsts = (d0, d1, d2, d3, d4, d5)
    ssems = (ss0, ss1, ss2, ss3, ss4, ss5)
    isems = (is0, is1, is2, is3, is4, is5)
    _zero_vmem(ones_v, B, 16)
    for st in ZSTARTS:
        pltpu.async_copy(ones_v, deg_sh.at[pl.ds(base + st, B)], ss0)
    for st in ZSTARTS:
        pltpu.make_async_copy(ones_v, deg_sh.at[pl.ds(base + st, B)],
                              ss0).wait()
    one = jnp.full((16,), 1.0, jnp.float32)

    def fill(r, carry):
        ones_v[r, pl.ds(0, 16)] = one
        return carry

    lax.fori_loop(0, B, fill, 0)
    plsc.subcore_barrier()

    first = (c * 16 + t) * WB

    def start_idx(k, u):
        pltpu.async_copy(dst_hbm.at[first + k], dsts[u], isems[u])

    def wait_idx(k, u):
        pltpu.make_async_copy(dst_hbm.at[first + k], dsts[u],
                              isems[u]).wait()

    def start_scatter(ki):
        pltpu.async_copy(ones_v, deg_sh.at[dsts[ki]], ssems[ki], add=True)

    def wait_scatter(ki):
        pltpu.make_async_copy(ones_v, deg_sh.at[dsts[ki]], ssems[ki]).wait()

    start_idx(0, 0)
    start_idx(1, 1)
    for k in range(4):
        wait_idx(k, k)
        start_scatter(k)
        start_idx(k + 2, k + 2)

    def step(k, i6):
        wait_idx(k, i6)
        wait_scatter((i6 + 2) % 6)
        start_scatter(i6)
        start_idx(k + 2, (i6 + 2) % 6)

    def body(j, carry):
        for u in range(6):
            step(4 + 6 * j + u, (4 + u) % 6)
        return carry

    lax.fori_loop(0, (WB - 8) // 6, body, 0)
    for kk in range(WB - 4, WB):
        step(kk, kk % 6)
    for kk in range(WB - 4, WB):
        wait_scatter(kk % 6)
    wait_idx(WB, WB % 6)
    wait_idx(WB + 1, (WB + 1) % 6)

    plsc.subcore_barrier()
    pltpu.sync_copy(deg_sh.at[pl.ds(base, RPT)],
                    deg_hbm.at[pl.ds(c * NP + base, RPT)])


@functools.partial(
    pl.kernel,
    out_type=jax.ShapeDtypeStruct((2 * NP, 128), jnp.float32),
    mesh=_MESH,
    scratch_types=(
        [pltpu.VMEM_SHARED((NP, 128), jnp.float32)]
        + [pltpu.VMEM((B,), jnp.int32)] * 12
        + [pltpu.VMEM((B, 128), jnp.float32)] * 3
        + [pltpu.SemaphoreType.DMA] * 12
    ),
)
def _edge_pass(g_hbm, src_hbm, dst_hbm, raw_hbm, acc_sh,
               s0, s1, s2, s3, s4, s5, d0, d1, d2, d3, d4, d5, rw0, rw1, rw2,
               gsem0, gsem1, gsem2, ssem0, ssem1, ssem2,
               is0, is1, is2, is3, is4, is5):
    c = lax.axis_index("c")
    t = lax.axis_index("s")
    base = t * RPT
    first = (c * 16 + t) * WB
    srcs = (s0, s1, s2, s3, s4, s5)
    dsts = (d0, d1, d2, d3, d4, d5)
    rows = (rw0, rw1, rw2)
    gsems = (gsem0, gsem1, gsem2)
    ssems = (ssem0, ssem1, ssem2)
    isems = (is0, is1, is2, is3, is4, is5)

    def start_idx(k, u):
        pltpu.async_copy(src_hbm.at[first + k], srcs[u], isems[u])
        pltpu.async_copy(dst_hbm.at[first + k], dsts[u], isems[u])

    def wait_idx(k, u):
        pltpu.make_async_copy(src_hbm.at[first + k], srcs[u],
                              isems[u]).wait()
        pltpu.make_async_copy(dst_hbm.at[first + k], dsts[u],
                              isems[u]).wait()

    def start_gather(ki, u):
        pltpu.async_copy(g_hbm.at[srcs[ki]], rows[u], gsems[u])

    def wait_gather(ki, u):
        pltpu.make_async_copy(g_hbm.at[srcs[ki]], rows[u],
                              gsems[u]).wait()

    def start_scatter(ki, u):
        pltpu.async_copy(rows[u], acc_sh.at[dsts[ki]], ssems[u],
                         add=True)

    def wait_scatter(ki, u):
        pltpu.make_async_copy(rows[u], acc_sh.at[dsts[ki]],
                              ssems[u]).wait()

    _zero_vmem(rw2, B, 128)
    for st in ZSTARTS:
        pltpu.async_copy(rw2, acc_sh.at[pl.ds(base + st, B)], ssem0)

    start_idx(0, 0)
    start_idx(1, 1)
    wait_idx(0, 0)
    start_gather(0, 0)
    start_idx(2, 2)
    wait_idx(1, 1)
    start_gather(1, 1)
    start_idx(3, 3)
    for st in ZSTARTS:
        pltpu.make_async_copy(rw2, acc_sh.at[pl.ds(base + st, B)],
                              ssem0).wait()
    plsc.subcore_barrier()
    wait_gather(0, 0)
    start_scatter(0, 0)

    def body(j, carry):
        for u in range(6):
            k = 2 + 6 * j + u
            i6 = (2 + u) % 6
            r3 = (2 + u) % 3
            wait_idx(k, i6)
            start_gather(i6, r3)
            start_idx(k + 2, (4 + u) % 6)
            wait_gather((1 + u) % 6, (1 + u) % 3)
            wait_scatter(u % 6, u % 3)
            start_scatter((1 + u) % 6, (1 + u) % 3)
        return carry

    lax.fori_loop(0, (WB - 2) // 6, body, 0)

    wait_gather((WB - 1) % 6, (WB - 1) % 3)
    wait_scatter((WB - 2) % 6, (WB - 2) % 3)
    start_scatter((WB - 1) % 6, (WB - 1) % 3)
    wait_scatter((WB - 1) % 6, (WB - 1) % 3)
    wait_idx(WB, WB % 6)
    wait_idx(WB + 1, (WB + 1) % 6)

    plsc.subcore_barrier()
    pltpu.sync_copy(acc_sh.at[pl.ds(base, RPT)],
                    raw_hbm.at[pl.ds(c * NP + base, RPT)])


def _dis_block(da_ref, db_ref):
    deg = da_ref[0, :, 0:1] + db_ref[0, :, 0:1] + 1.0
    return lax.rsqrt(deg)


def _tc_first(x, W1, deg3d):
    def body(x_ref, w_ref, da_ref, db_ref, o_ref):
        dis = _dis_block(da_ref, db_ref)
        h = jnp.dot(x_ref[...], w_ref[...], preferred_element_type=jnp.float32)
        o_ref[...] = dis * h

    nb = N // BN
    return pl.pallas_call(
        body,
        grid=(nb,),
        in_specs=[
            pl.BlockSpec((BN, 128), lambda i: (i, 0)),
            pl.BlockSpec((128, 128), lambda i: (0, 0)),
            pl.BlockSpec((1, BN, 16), lambda i: (0, i, 0)),
            pl.BlockSpec((1, BN, 16), lambda i: (1, i, 0)),
        ],
        out_specs=pl.BlockSpec((BN, 128), lambda i: (i, 0)),
        out_shape=jax.ShapeDtypeStruct((N, 128), jnp.float32),
    )(x, W1, deg3d, deg3d)


def _tc_mid(raw3d, g, deg3d, b2d, W):
    fo = W.shape[1]

    def body(ra_ref, rb_ref, g_ref, da_ref, db_ref, b_ref, w_ref, o_ref):
        dis = _dis_block(da_ref, db_ref)
        z = dis * (ra_ref[0] + rb_ref[0] + g_ref[...]) + b_ref[...]
        a = jnp.maximum(z, 0.0)
        h = jnp.dot(a, w_ref[...], preferred_element_type=jnp.float32)
        h = dis * h
        if fo < 128:
            h = jnp.concatenate(
                [h, jnp.zeros((h.shape[0], 128 - fo), jnp.float32)], axis=1)
        o_ref[...] = h

    nb = N // BN
    return pl.pallas_call(
        body,
        grid=(nb,),
        in_specs=[
            pl.BlockSpec((1, BN, 128), lambda i: (0, i, 0)),
            pl.BlockSpec((1, BN, 128), lambda i: (1, i, 0)),
            pl.BlockSpec((BN, 128), lambda i: (i, 0)),
            pl.BlockSpec((1, BN, 16), lambda i: (0, i, 0)),
            pl.BlockSpec((1, BN, 16), lambda i: (1, i, 0)),
            pl.BlockSpec((1, 128), lambda i: (0, 0)),
            pl.BlockSpec((128, fo), lambda i: (0, 0)),
        ],
        out_specs=pl.BlockSpec((BN, 128), lambda i: (i, 0)),
        out_shape=jax.ShapeDtypeStruct((N, 128), jnp.float32),
    )(raw3d, raw3d, g, deg3d, deg3d, b2d, W)


def _tc_final(raw3d, g, deg3d, b2):
    def body(ra_ref, rb_ref, g_ref, da_ref, db_ref, b_ref, o_ref):
        dis = _dis_block(da_ref, db_ref)
        z = dis * (ra_ref[0, :, 0:64] + rb_ref[0, :, 0:64] + g_ref[:, 0:64])
        z = z + b_ref[...]
        m = jnp.max(z, axis=1, keepdims=True)
        s = z - m
        o_ref[...] = s - jnp.log(jnp.sum(jnp.exp(s), axis=1, keepdims=True))

    nb = N // BN
    return pl.pallas_call(
        body,
        grid=(nb,),
        in_specs=[
            pl.BlockSpec((1, BN, 128), lambda i: (0, i, 0)),
            pl.BlockSpec((1, BN, 128), lambda i: (1, i, 0)),
            pl.BlockSpec((BN, 128), lambda i: (i, 0)),
            pl.BlockSpec((1, BN, 16), lambda i: (0, i, 0)),
            pl.BlockSpec((1, BN, 16), lambda i: (1, i, 0)),
            pl.BlockSpec((1, 64), lambda i: (0, 0)),
        ],
        out_specs=pl.BlockSpec((BN, 64), lambda i: (i, 0)),
        out_shape=jax.ShapeDtypeStruct((N, 64), jnp.float32),
    )(raw3d, raw3d, g, deg3d, deg3d, b2)


def kernel(x, edge_index, W1, b1, Wg0, bg0, Wg1, bg1, W2, b2):
    src = edge_index[0]
    dst = edge_index[1]
    pad = E_PAD - E + 2 * B
    ar = jnp.arange(pad, dtype=jnp.int32)
    srcp = jnp.concatenate([src, ar % N]).reshape(NBATCH + 2, B)
    dstp = jnp.concatenate([dst, N + (ar % 112)]).reshape(NBATCH + 2, B)

    deg3d = _deg_kernel(dstp).reshape(2, NP, 16)

    g1 = _tc_first(x, W1, deg3d)
    raw1 = _edge_pass(g1, srcp, dstp).reshape(2, NP, 128)
    g2 = _tc_mid(raw1, g1, deg3d, b1.reshape(1, 128), Wg0)
    raw2 = _edge_pass(g2, srcp, dstp).reshape(2, NP, 128)
    g3 = _tc_mid(raw2, g2, deg3d, bg0.reshape(1, 128), Wg1)
    raw3 = _edge_pass(g3, srcp, dstp).reshape(2, NP, 128)
    g4 = _tc_mid(raw3, g3, deg3d, bg1.reshape(1, 128), W2)
    raw4 = _edge_pass(g4, srcp, dstp).reshape(2, NP, 128)
    return _tc_final(raw4, g4, deg3d, b2.reshape(1, 64))
